# baseline (speedup 1.0000x reference)
import functools

import jax
import jax.numpy as jnp
from jax import lax
from jax.experimental import pallas as pl
from jax.experimental.pallas import tpu as pltpu
from jax.experimental.pallas import tpu_sc as plsc

N_NODE = 10000
N_EDGE = 160000
H = 128
D_E = 32
CUTOFF = 10.0

CH = 128
NCHUNK = N_EDGE // CH
NSUB = 16
ITERS = -(-NCHUNK // NSUB)
HALF = 5120
NR = 5248
RPT = NR // NSUB

f32 = jnp.float32
i32 = jnp.int32


def _node_proj_body(hl, hp, wcat, al, bl, cl, dl, ap, bp, cp, dp):
    rl = jnp.dot(hl[...], wcat[...], preferred_element_type=f32)
    rp = jnp.dot(hp[...], wcat[...], preferred_element_type=f32)
    al[...] = rl[:, 0:128]
    bl[...] = rl[:, 128:256]
    cl[...] = rl[:, 256:384]
    dl[...] = rl[:, 384:512]
    ap[...] = rp[:, 0:128]
    bp[...] = rp[:, 128:256]
    cp[...] = rp[:, 256:384]
    dp[...] = rp[:, 384:512]


def _node_proj(h_l, h_p, wcat):
    blk = 1000
    grid = N_NODE // blk
    node_spec = pl.BlockSpec((blk, H), lambda i: (i, 0))
    return pl.pallas_call(
        _node_proj_body,
        grid=(grid,),
        in_specs=[node_spec, node_spec,
                  pl.BlockSpec((H, 4 * H), lambda i: (0, 0))],
        out_specs=[node_spec] * 8,
        out_shape=[jax.ShapeDtypeStruct((N_NODE, H), f32)] * 8,
    )(h_l, h_p, wcat)


def _edge_pre_body(a_ll, a_pp, a_pl, w_ll, w_pp, w_pl, wbi, wbt, b1i, b1t,
                   p_ll, p_pp, q_pl, fs_ll, fs_pp, fs_pl):
    p_ll[...] = jnp.dot(a_ll[...], wbi[...], preferred_element_type=f32) + b1i[...]
    p_pp[...] = jnp.dot(a_pp[...], wbi[...], preferred_element_type=f32) + b1i[...]
    q_pl[...] = jnp.dot(a_pl[...], wbt[...], preferred_element_type=f32) + b1t[...]

    zeros = jnp.zeros(w_ll.shape + (16,), f32)

    def filt_splat(w):
        f = 0.5 * (jnp.cos(w[...] * (jnp.pi / CUTOFF)) + 1.0)
        return f[..., None] + zeros

    fs_ll[...] = filt_splat(w_ll)
    fs_pp[...] = filt_splat(w_pp)
    fs_pl[...] = filt_splat(w_pl)


def _edge_pre(attr_ll, attr_pp, attr_pl, w_ll, w_pp, w_pl, wbi, wbt, b1i, b1t):
    eblk = 3200
    grid = N_EDGE // eblk
    wrows = eblk // 128
    attr_spec = pl.BlockSpec((eblk, D_E), lambda i: (i, 0))
    w_spec = pl.BlockSpec((1, wrows, 128), lambda i: (i, 0, 0))
    p_spec = pl.BlockSpec((eblk, H), lambda i: (i, 0))
    fs_spec = pl.BlockSpec((1, wrows, 128, 16), lambda i: (i, 0, 0, 0))
    wb_spec = pl.BlockSpec((D_E, H), lambda i: (0, 0))
    b_spec = pl.BlockSpec((1, H), lambda i: (0, 0))
    outs = pl.pallas_call(
        _edge_pre_body,
        grid=(grid,),
        in_specs=[attr_spec, attr_spec, attr_spec, w_spec, w_spec, w_spec,
                  wb_spec, wb_spec, b_spec, b_spec],
        out_specs=[p_spec] * 3 + [fs_spec] * 3,
        out_shape=[jax.ShapeDtypeStruct((N_EDGE, H), f32)] * 3
        + [jax.ShapeDtypeStruct((grid, wrows, 128, 16), f32)] * 3,
    )(attr_ll, attr_pp, attr_pl,
      w_ll.reshape(grid, wrows, 128), w_pp.reshape(grid, wrows, 128),
      w_pl.reshape(grid, wrows, 128), wbi, wbt, b1i, b1t)
    return tuple(outs[:3]) + tuple(o.reshape(N_EDGE, 16) for o in outs[3:])


def _zero_rows(src_v, sh, base, nrows):
    done = 0
    while done < nrows:
        n = min(CH, nrows - done)
        pltpu.sync_copy(src_v.at[pl.ds(0, n)], sh.at[pl.ds(base + done, n)])
        done += n


def _copy_rows(sh, dst, base, nrows):
    done = 0
    while done < nrows:
        n = min(CH, nrows - done)
        pltpu.sync_copy(sh.at[pl.ds(base + done, n)],
                        dst.at[pl.ds(base + done, n)])
        done += n


def _route_idx(ib_v, ib2_v, base):
    for g in range(CH // 16):
        s = pl.ds(g * 16, 16)
        iv = ib_v[0, s]
        loc = iv - base
        ok = jnp.logical_and(loc >= 0, loc < HALF)
        ib2_v[0, s] = jnp.where(ok, loc, HALF)


def _sc_edge_body(ta, tb, p, fsin, ia, ib,
                  u0, u1,
                  pay_v, a_v, b_v, fs_v, ia_v, ib_v, ib2_v, ush):
    cid = lax.axis_index("c")
    sid = lax.axis_index("s")
    base_nodes = cid * HALF

    def zrow(j, _):
        zero = jnp.zeros((16,), f32)
        for r in range(H // 16):
            pay_v[j, pl.ds(r * 16, 16)] = zero
        return 0

    lax.fori_loop(0, CH, zrow, 0)
    _zero_rows(pay_v, ush, sid * RPT, RPT)
    plsc.subcore_barrier()

    def chunk_body(it, _):
        chunk = sid + it * NSUB

        @pl.when(chunk < NCHUNK)
        def _():
            base = chunk * CH
            pltpu.sync_copy(ia.at[pl.ds(base, CH)], ia_v.at[0])
            pltpu.sync_copy(ib.at[pl.ds(base, CH)], ib_v.at[0])
            pltpu.sync_copy(p.at[pl.ds(base, CH)], pay_v)
            pltpu.sync_copy(fsin.at[pl.ds(base, CH)], fs_v)
            pltpu.sync_copy(ta.at[ia_v.at[0]], a_v)
            pltpu.sync_copy(tb.at[ib_v.at[0]], b_v)
            _route_idx(ib_v, ib2_v, base_nodes)

            def e_body(j, _):
                fv = fs_v[j, pl.ds(0, 16)]
                for r in range(H // 16):
                    s = pl.ds(r * 16, 16)
                    z = pay_v[j, s] + a_v[j, s] + b_v[j, s]
                    u = (z / (1.0 + jnp.exp(-z))) * fv
                    pay_v[j, s] = u
                return 0

            lax.fori_loop(0, CH, e_body, 0)
            pltpu.sync_copy(pay_v, ush.at[ib2_v.at[0]], add=True)

        return 0

    lax.fori_loop(0, ITERS, chunk_body, 0)
    plsc.subcore_barrier()

    @pl.when(cid == 0)
    def _():
        _copy_rows(ush, u0, sid * RPT, RPT)

    @pl.when(cid == 1)
    def _():
        _copy_rows(ush, u1, sid * RPT, RPT)


@functools.partial(
    pl.kernel,
    mesh=plsc.VectorSubcoreMesh(core_axis_name="c", subcore_axis_name="s"),
    out_type=[jax.ShapeDtypeStruct((NR, H), f32),
              jax.ShapeDtypeStruct((NR, H), f32)],
    scratch_types=[pltpu.VMEM((CH, H), f32),
                   pltpu.VMEM((CH, H), f32),
                   pltpu.VMEM((CH, H), f32),
                   pltpu.VMEM((CH, 16), f32),
                   pltpu.VMEM((1, CH), i32),
                   pltpu.VMEM((1, CH), i32),
                   pltpu.VMEM((1, CH), i32),
                   pltpu.VMEM_SHARED((NR, H), f32)],
)
def _sc_edge_pass(*refs):
    _sc_edge_body(*refs)


def _sc_fc_body(fs0, ib0, fs1, ib1, fs2, ib2, fs3, ib3,
                o00, o01, o10, o11, o20, o21, o30, o31,
                pay_v, fs_v, ib_v, ib2_v, ush):
    cid = lax.axis_index("c")
    sid = lax.axis_index("s")
    base_nodes = cid * HALF
    lane = lax.iota(i32, 16)

    def zrow(j, _):
        zero = jnp.zeros((16,), f32)
        for r in range(H // 16):
            pay_v[j, pl.ds(r * 16, 16)] = zero
        return 0

    lax.fori_loop(0, CH, zrow, 0)

    for fsin, ibin, out0, out1 in ((fs0, ib0, o00, o01), (fs1, ib1, o10, o11),
                                   (fs2, ib2, o20, o21), (fs3, ib3, o30, o31)):
        def zfirst(j, _):
            pay_v[j, pl.ds(0, 16)] = jnp.zeros((16,), f32)
            return 0

        lax.fori_loop(0, CH, zfirst, 0)
        _zero_rows(pay_v, ush, sid * RPT, RPT)
        plsc.subcore_barrier()

        def chunk_body(it, _):
            chunk = sid + it * NSUB

            @pl.when(chunk < NCHUNK)
            def _():
                base = chunk * CH
                pltpu.sync_copy(ibin.at[pl.ds(base, CH)], ib_v.at[0])
                pltpu.sync_copy(fsin.at[pl.ds(base, CH)], fs_v)
                _route_idx(ib_v, ib2_v, base_nodes)

                def e_body(j, _):
                    fv = fs_v[j, pl.ds(0, 16)]
                    row = jnp.where(lane == 0, fv,
                                    jnp.where(lane == 1, 1.0, 0.0))
                    pay_v[j, pl.ds(0, 16)] = row
                    return 0

                lax.fori_loop(0, CH, e_body, 0)
                pltpu.sync_copy(pay_v, ush.at[ib2_v.at[0]], add=True)

            return 0

        lax.fori_loop(0, ITERS, chunk_body, 0)
        plsc.subcore_barrier()

        @pl.when(cid == 0)
        def _():
            _copy_rows(ush, out0, sid * RPT, RPT)

        @pl.when(cid == 1)
        def _():
            _copy_rows(ush, out1, sid * RPT, RPT)


@functools.partial(
    pl.kernel,
    mesh=plsc.VectorSubcoreMesh(core_axis_name="c", subcore_axis_name="s"),
    out_type=[jax.ShapeDtypeStruct((NR, H), f32)] * 8,
    scratch_types=[pltpu.VMEM((CH, H), f32),
                   pltpu.VMEM((CH, 16), f32),
                   pltpu.VMEM((1, CH), i32),
                   pltpu.VMEM((1, CH), i32),
                   pltpu.VMEM_SHARED((NR, H), f32)],
)
def _sc_fc_pass(*refs):
    _sc_fc_body(*refs)


def _node_post_body(u_ll, fc_ll, u_pl, fc_pl, u_pp, fc_pp, u_lp, fc_lp,
                    hl, hp, w2i, w2t, b2i, b2t, gwa, gwb, gb,
                    wih_l, whh_l, bih_l, bhh_l, wih_p, whh_p, bih_p, bhh_p,
                    out_l, out_p):
    def agg(u, fc, w2, b2):
        fsum = fc[:, 0:1]
        cnt = fc[:, 1:2]
        num = (jnp.dot(u[...], w2[...], preferred_element_type=f32)
               + b2[...] * fsum)
        return num / jnp.maximum(cnt, 1.0)

    a_ll = agg(u_ll, fc_ll, w2i, b2i)
    a_pl = agg(u_pl, fc_pl, w2t, b2t)
    a_pp = agg(u_pp, fc_pp, w2i, b2i)
    a_lp = agg(u_lp, fc_lp, w2t, b2t)

    def side(a_main, a_cross, h, wih, whh, bih, bhh, out):
        g = jax.nn.sigmoid(
            jnp.dot(a_main, gwa[...], preferred_element_type=f32)
            + jnp.dot(a_cross, gwb[...], preferred_element_type=f32) + gb[...])
        msg = g * a_main + (1.0 - g) * a_cross
        gi = jnp.dot(h, wih[...], preferred_element_type=f32) + bih[...]
        gh = jnp.dot(msg, whh[...], preferred_element_type=f32) + bhh[...]
        r = jax.nn.sigmoid(gi[:, 0:128] + gh[:, 0:128])
        zg = jax.nn.sigmoid(gi[:, 128:256] + gh[:, 128:256])
        n = jnp.tanh(gi[:, 256:384] + r * gh[:, 256:384])
        out[...] = (1.0 - zg) * n + zg * msg

    side(a_ll, a_pl, hl[...], wih_l, whh_l, bih_l, bhh_l, out_l)
    side(a_pp, a_lp, hp[...], wih_p, whh_p, bih_p, bhh_p, out_p)


def _node_post(u_ll, fc_ll, u_pl, fc_pl, u_pp, fc_pp, u_lp, fc_lp,
               h_l, h_p, w2i, w2t, b2i, b2t, gwa, gwb, gb,
               wih_l, whh_l, bih_l, bhh_l, wih_p, whh_p, bih_p, bhh_p):
    blk = 1000
    grid = N_NODE // blk
    n_spec = pl.BlockSpec((blk, H), lambda i: (i, 0))
    w_spec = pl.BlockSpec((H, H), lambda i: (0, 0))
    b_spec = pl.BlockSpec((1, H), lambda i: (0, 0))
    gw_spec = pl.BlockSpec((H, 1), lambda i: (0, 0))
    gb_spec = pl.BlockSpec((1, 1), lambda i: (0, 0))
    gru_w_spec = pl.BlockSpec((H, 3 * H), lambda i: (0, 0))
    gru_b_spec = pl.BlockSpec((1, 3 * H), lambda i: (0, 0))
    return pl.pallas_call(
        _node_post_body,
        grid=(grid,),
        in_specs=[n_spec] * 10
        + [w_spec, w_spec, b_spec, b_spec, gw_spec, gw_spec, gb_spec,
           gru_w_spec, gru_w_spec, gru_b_spec, gru_b_spec,
           gru_w_spec, gru_w_spec, gru_b_spec, gru_b_spec],
        out_specs=[n_spec, n_spec],
        out_shape=[jax.ShapeDtypeStruct((N_NODE, H), f32)] * 2,
    )(u_ll, fc_ll, u_pl, fc_pl, u_pp, fc_pp, u_lp, fc_lp,
      h_l, h_p, w2i, w2t, b2i, b2t, gwa, gwb, gb,
      wih_l, whh_l, bih_l, bhh_l, wih_p, whh_p, bih_p, bhh_p)


def _merge_halves(lo, hi):
    return jnp.concatenate([lo[:HALF], hi[:N_NODE - HALF]], axis=0)


def kernel(h_l, h_p, edge_index_ll, edge_attr_ll, edge_weight_ll,
           edge_index_pp, edge_attr_pp, edge_weight_pp,
           edge_index_pl, edge_attr_pl, edge_weight_pl,
           inter_W1, inter_b1, inter_W2, inter_b2,
           intra_W1, intra_b1, intra_W2, intra_b2,
           gate_W, gate_b,
           gru_l_Wih, gru_l_Whh, gru_l_bih, gru_l_bhh,
           gru_p_Wih, gru_p_Whh, gru_p_bih, gru_p_bhh):
    wcat = jnp.concatenate([inter_W1[0:H], inter_W1[H + D_E:],
                            intra_W1[0:H], intra_W1[H + D_E:]], axis=1)
    wbi = inter_W1[H:H + D_E]
    wbt = intra_W1[H:H + D_E]
    b1i = inter_b1.reshape(1, H)
    b1t = intra_b1.reshape(1, H)

    a_l, b_l, c_l, d_l, a_p, b_p, c_p, d_p = _node_proj(h_l, h_p, wcat)
    p_ll, p_pp, q_pl, fs_ll, fs_pp, fs_pl = _edge_pre(
        edge_attr_ll, edge_attr_pp, edge_attr_pl,
        edge_weight_ll, edge_weight_pp, edge_weight_pl, wbi, wbt, b1i, b1t)

    ll0 = edge_index_ll[0].astype(i32)
    ll1 = edge_index_ll[1].astype(i32)
    pp0 = edge_index_pp[0].astype(i32)
    pp1 = edge_index_pp[1].astype(i32)
    pl0 = edge_index_pl[0].astype(i32)
    pl1 = edge_index_pl[1].astype(i32)

    u_ll = _merge_halves(*_sc_edge_pass(a_l, b_l, p_ll, fs_ll, ll0, ll1))
    u_pp = _merge_halves(*_sc_edge_pass(a_p, b_p, p_pp, fs_pp, pp0, pp1))
    u_pl = _merge_halves(*_sc_edge_pass(c_p, d_l, q_pl, fs_pl, pl0, pl1))
    u_lp = _merge_halves(*_sc_edge_pass(c_l, d_p, q_pl, fs_pl, pl1, pl0))

    fc_out = _sc_fc_pass(fs_ll, ll1, fs_pp, pp1, fs_pl, pl1, fs_pl, pl0)
    fc_ll = _merge_halves(fc_out[0], fc_out[1])
    fc_pp = _merge_halves(fc_out[2], fc_out[3])
    fc_pl = _merge_halves(fc_out[4], fc_out[5])
    fc_lp = _merge_halves(fc_out[6], fc_out[7])

    new_h_l, new_h_p = _node_post(
        u_ll, fc_ll, u_pl, fc_pl, u_pp, fc_pp, u_lp, fc_lp,
        h_l, h_p, inter_W2, intra_W2,
        inter_b2.reshape(1, H), intra_b2.reshape(1, H),
        gate_W[0:H], gate_W[H:], gate_b.reshape(1, 1),
        gru_l_Wih.T, gru_l_Whh.T, gru_l_bih.reshape(1, 3 * H),
        gru_l_bhh.reshape(1, 3 * H),
        gru_p_Wih.T, gru_p_Whh.T, gru_p_bih.reshape(1, 3 * H),
        gru_p_bhh.reshape(1, 3 * H))
    return (new_h_l, new_h_p)


# speedup vs baseline: 1.0519x; 1.0519x over previous
import functools

import jax
import jax.numpy as jnp
from jax import lax
from jax.experimental import pallas as pl
from jax.experimental.pallas import tpu as pltpu
from jax.experimental.pallas import tpu_sc as plsc

N_NODE = 10000
N_EDGE = 160000
H = 128
D_E = 32
CUTOFF = 10.0

CH = 128
NCHUNK = N_EDGE // CH
NSUB = 16
ITERS = -(-NCHUNK // NSUB)
HALF = 5120
NR = 5248
RPT = NR // NSUB

f32 = jnp.float32
i32 = jnp.int32


def _node_proj_body(hl, hp, wcat, al, bl, cl, dl, ap, bp, cp, dp):
    rl = jnp.dot(hl[...], wcat[...], preferred_element_type=f32)
    rp = jnp.dot(hp[...], wcat[...], preferred_element_type=f32)
    al[...] = rl[:, 0:128]
    bl[...] = rl[:, 128:256]
    cl[...] = rl[:, 256:384]
    dl[...] = rl[:, 384:512]
    ap[...] = rp[:, 0:128]
    bp[...] = rp[:, 128:256]
    cp[...] = rp[:, 256:384]
    dp[...] = rp[:, 384:512]


def _node_proj(h_l, h_p, wcat):
    blk = 1000
    grid = N_NODE // blk
    node_spec = pl.BlockSpec((blk, H), lambda i: (i, 0))
    return pl.pallas_call(
        _node_proj_body,
        grid=(grid,),
        in_specs=[node_spec, node_spec,
                  pl.BlockSpec((H, 4 * H), lambda i: (0, 0))],
        out_specs=[node_spec] * 8,
        out_shape=[jax.ShapeDtypeStruct((N_NODE, H), f32)] * 8,
    )(h_l, h_p, wcat)


def _edge_pre_body(a_ll, a_pp, a_pl, w_ll, w_pp, w_pl, wbi, wbt, b1i, b1t,
                   p_ll, p_pp, q_pl, fs_ll, fs_pp, fs_pl):
    p_ll[...] = jnp.dot(a_ll[...], wbi[...], preferred_element_type=f32) + b1i[...]
    p_pp[...] = jnp.dot(a_pp[...], wbi[...], preferred_element_type=f32) + b1i[...]
    q_pl[...] = jnp.dot(a_pl[...], wbt[...], preferred_element_type=f32) + b1t[...]

    zeros = jnp.zeros(w_ll.shape + (16,), f32)

    def filt_splat(w):
        f = 0.5 * (jnp.cos(w[...] * (jnp.pi / CUTOFF)) + 1.0)
        return f[..., None] + zeros

    fs_ll[...] = filt_splat(w_ll)
    fs_pp[...] = filt_splat(w_pp)
    fs_pl[...] = filt_splat(w_pl)


def _edge_pre(attr_ll, attr_pp, attr_pl, w_ll, w_pp, w_pl, wbi, wbt, b1i, b1t):
    eblk = 3200
    grid = N_EDGE // eblk
    wrows = eblk // 128
    attr_spec = pl.BlockSpec((eblk, D_E), lambda i: (i, 0))
    w_spec = pl.BlockSpec((1, wrows, 128), lambda i: (i, 0, 0))
    p_spec = pl.BlockSpec((eblk, H), lambda i: (i, 0))
    fs_spec = pl.BlockSpec((1, wrows, 128, 16), lambda i: (i, 0, 0, 0))
    wb_spec = pl.BlockSpec((D_E, H), lambda i: (0, 0))
    b_spec = pl.BlockSpec((1, H), lambda i: (0, 0))
    outs = pl.pallas_call(
        _edge_pre_body,
        grid=(grid,),
        in_specs=[attr_spec, attr_spec, attr_spec, w_spec, w_spec, w_spec,
                  wb_spec, wb_spec, b_spec, b_spec],
        out_specs=[p_spec] * 3 + [fs_spec] * 3,
        out_shape=[jax.ShapeDtypeStruct((N_EDGE, H), f32)] * 3
        + [jax.ShapeDtypeStruct((grid, wrows, 128, 16), f32)] * 3,
    )(attr_ll, attr_pp, attr_pl,
      w_ll.reshape(grid, wrows, 128), w_pp.reshape(grid, wrows, 128),
      w_pl.reshape(grid, wrows, 128), wbi, wbt, b1i, b1t)
    return tuple(outs[:3]) + tuple(o.reshape(N_EDGE, 16) for o in outs[3:])


def _zero_rows(src_v, sh, base, nrows):
    done = 0
    while done < nrows:
        n = min(CH, nrows - done)
        pltpu.sync_copy(src_v.at[pl.ds(0, n)], sh.at[pl.ds(base + done, n)])
        done += n


def _copy_rows(sh, dst, base, nrows):
    done = 0
    while done < nrows:
        n = min(CH, nrows - done)
        pltpu.sync_copy(sh.at[pl.ds(base + done, n)],
                        dst.at[pl.ds(base + done, n)])
        done += n


def _route_idx(ib_v, slot, ib2_v, base):
    for g in range(CH // 16):
        s = pl.ds(g * 16, 16)
        iv = ib_v[slot, s]
        loc = iv - base
        ok = jnp.logical_and(loc >= 0, loc < HALF)
        ib2_v[0, s] = jnp.where(ok, loc, HALF)


def _sc_edge_body(ta, tb, p, fsin, ia, ib,
                  u0, u1,
                  p_v, a_v, b_v, fs_v, ia_v, ib_v, ib2_v, ush):
    cid = lax.axis_index("c")
    sid = lax.axis_index("s")
    base_nodes = cid * HALF

    def zrow(j, _):
        zero = jnp.zeros((16,), f32)
        for r in range(H // 16):
            a_v[j, pl.ds(r * 16, 16)] = zero
        return 0

    lax.fori_loop(0, CH, zrow, 0)
    _zero_rows(a_v, ush, sid * RPT, RPT)
    plsc.subcore_barrier()

    def chunk_body(it, _):
        chunk = sid + it * NSUB

        @pl.when(chunk < NCHUNK)
        def _():
            base = chunk * CH
            pltpu.sync_copy(ia.at[pl.ds(base, CH)], ia_v.at[0])
            pltpu.sync_copy(ib.at[pl.ds(base, CH)], ib_v.at[0])
            pltpu.sync_copy(p.at[pl.ds(base, CH)], p_v)
            pltpu.sync_copy(fsin.at[pl.ds(base, CH)], fs_v)
            pltpu.sync_copy(ta.at[ia_v.at[0]], a_v)
            pltpu.sync_copy(tb.at[ib_v.at[0]], b_v)
            _route_idx(ib_v, 0, ib2_v, base_nodes)

            def e_body(j2, _):
                for j in (j2 * 2, j2 * 2 + 1):
                    fv = fs_v[j, pl.ds(0, 16)]
                    for r in range(H // 16):
                        s = pl.ds(r * 16, 16)
                        z = p_v[j, s] + a_v[j, s] + b_v[j, s]
                        u = (z / (1.0 + jnp.exp(-z))) * fv
                        p_v[j, s] = u
                return 0

            lax.fori_loop(0, CH // 2, e_body, 0)
            pltpu.sync_copy(p_v, ush.at[ib2_v.at[0]], add=True)

        return 0

    lax.fori_loop(0, ITERS, chunk_body, 0)
    plsc.subcore_barrier()

    @pl.when(cid == 0)
    def _():
        _copy_rows(ush, u0, sid * RPT, RPT)

    @pl.when(cid == 1)
    def _():
        _copy_rows(ush, u1, sid * RPT, RPT)


@functools.partial(
    pl.kernel,
    mesh=plsc.VectorSubcoreMesh(core_axis_name="c", subcore_axis_name="s"),
    out_type=[jax.ShapeDtypeStruct((NR, H), f32),
              jax.ShapeDtypeStruct((NR, H), f32)],
    scratch_types=[pltpu.VMEM((CH, H), f32),
                   pltpu.VMEM((CH, H), f32),
                   pltpu.VMEM((CH, H), f32),
                   pltpu.VMEM((CH, 16), f32),
                   pltpu.VMEM((1, CH), i32),
                   pltpu.VMEM((1, CH), i32),
                   pltpu.VMEM((1, CH), i32),
                   pltpu.VMEM_SHARED((NR, H), f32)],
)
def _sc_edge_pass(*refs):
    _sc_edge_body(*refs)


def _sc_fc_body(fs0, ib0, fs1, ib1, fs2, ib2, fs3, ib3,
                o00, o01, o10, o11, o20, o21, o30, o31,
                pay_v, fs_v, ib_v, ib2_v, ush):
    cid = lax.axis_index("c")
    sid = lax.axis_index("s")
    base_nodes = cid * HALF
    lane = lax.iota(i32, 16)

    def zrow(j, _):
        zero = jnp.zeros((16,), f32)
        for r in range(H // 16):
            pay_v[j, pl.ds(r * 16, 16)] = zero
        return 0

    lax.fori_loop(0, CH, zrow, 0)

    for fsin, ibin, out0, out1 in ((fs0, ib0, o00, o01), (fs1, ib1, o10, o11),
                                   (fs2, ib2, o20, o21), (fs3, ib3, o30, o31)):
        def zfirst(j, _):
            pay_v[j, pl.ds(0, 16)] = jnp.zeros((16,), f32)
            return 0

        lax.fori_loop(0, CH, zfirst, 0)
        _zero_rows(pay_v, ush, sid * RPT, RPT)
        plsc.subcore_barrier()

        def chunk_body(it, _):
            chunk = sid + it * NSUB

            @pl.when(chunk < NCHUNK)
            def _():
                base = chunk * CH
                pltpu.sync_copy(ibin.at[pl.ds(base, CH)], ib_v.at[0])
                pltpu.sync_copy(fsin.at[pl.ds(base, CH)], fs_v)
                _route_idx(ib_v, 0, ib2_v, base_nodes)

                def e_body(j, _):
                    fv = fs_v[j, pl.ds(0, 16)]
                    row = jnp.where(lane == 0, fv,
                                    jnp.where(lane == 1, 1.0, 0.0))
                    pay_v[j, pl.ds(0, 16)] = row
                    return 0

                lax.fori_loop(0, CH, e_body, 0)
                pltpu.sync_copy(pay_v, ush.at[ib2_v.at[0]], add=True)

            return 0

        lax.fori_loop(0, ITERS, chunk_body, 0)
        plsc.subcore_barrier()

        @pl.when(cid == 0)
        def _():
            _copy_rows(ush, out0, sid * RPT, RPT)

        @pl.when(cid == 1)
        def _():
            _copy_rows(ush, out1, sid * RPT, RPT)


@functools.partial(
    pl.kernel,
    mesh=plsc.VectorSubcoreMesh(core_axis_name="c", subcore_axis_name="s"),
    out_type=[jax.ShapeDtypeStruct((NR, H), f32)] * 8,
    scratch_types=[pltpu.VMEM((CH, H), f32),
                   pltpu.VMEM((CH, 16), f32),
                   pltpu.VMEM((1, CH), i32),
                   pltpu.VMEM((1, CH), i32),
                   pltpu.VMEM_SHARED((NR, H), f32)],
)
def _sc_fc_pass(*refs):
    _sc_fc_body(*refs)


def _node_post_body(u_ll, fc_ll, u_pl, fc_pl, u_pp, fc_pp, u_lp, fc_lp,
                    hl, hp, w2i, w2t, b2i, b2t, gwa, gwb, gb,
                    wih_l, whh_l, bih_l, bhh_l, wih_p, whh_p, bih_p, bhh_p,
                    out_l, out_p):
    def agg(u, fc, w2, b2):
        fsum = fc[:, 0:1]
        cnt = fc[:, 1:2]
        num = (jnp.dot(u[...], w2[...], preferred_element_type=f32)
               + b2[...] * fsum)
        return num / jnp.maximum(cnt, 1.0)

    a_ll = agg(u_ll, fc_ll, w2i, b2i)
    a_pl = agg(u_pl, fc_pl, w2t, b2t)
    a_pp = agg(u_pp, fc_pp, w2i, b2i)
    a_lp = agg(u_lp, fc_lp, w2t, b2t)

    def side(a_main, a_cross, h, wih, whh, bih, bhh, out):
        g = jax.nn.sigmoid(
            jnp.dot(a_main, gwa[...], preferred_element_type=f32)
            + jnp.dot(a_cross, gwb[...], preferred_element_type=f32) + gb[...])
        msg = g * a_main + (1.0 - g) * a_cross
        gi = jnp.dot(h, wih[...], preferred_element_type=f32) + bih[...]
        gh = jnp.dot(msg, whh[...], preferred_element_type=f32) + bhh[...]
        r = jax.nn.sigmoid(gi[:, 0:128] + gh[:, 0:128])
        zg = jax.nn.sigmoid(gi[:, 128:256] + gh[:, 128:256])
        n = jnp.tanh(gi[:, 256:384] + r * gh[:, 256:384])
        out[...] = (1.0 - zg) * n + zg * msg

    side(a_ll, a_pl, hl[...], wih_l, whh_l, bih_l, bhh_l, out_l)
    side(a_pp, a_lp, hp[...], wih_p, whh_p, bih_p, bhh_p, out_p)


def _node_post(u_ll, fc_ll, u_pl, fc_pl, u_pp, fc_pp, u_lp, fc_lp,
               h_l, h_p, w2i, w2t, b2i, b2t, gwa, gwb, gb,
               wih_l, whh_l, bih_l, bhh_l, wih_p, whh_p, bih_p, bhh_p):
    blk = 1000
    grid = N_NODE // blk
    n_spec = pl.BlockSpec((blk, H), lambda i: (i, 0))
    w_spec = pl.BlockSpec((H, H), lambda i: (0, 0))
    b_spec = pl.BlockSpec((1, H), lambda i: (0, 0))
    gw_spec = pl.BlockSpec((H, 1), lambda i: (0, 0))
    gb_spec = pl.BlockSpec((1, 1), lambda i: (0, 0))
    gru_w_spec = pl.BlockSpec((H, 3 * H), lambda i: (0, 0))
    gru_b_spec = pl.BlockSpec((1, 3 * H), lambda i: (0, 0))
    return pl.pallas_call(
        _node_post_body,
        grid=(grid,),
        in_specs=[n_spec] * 10
        + [w_spec, w_spec, b_spec, b_spec, gw_spec, gw_spec, gb_spec,
           gru_w_spec, gru_w_spec, gru_b_spec, gru_b_spec,
           gru_w_spec, gru_w_spec, gru_b_spec, gru_b_spec],
        out_specs=[n_spec, n_spec],
        out_shape=[jax.ShapeDtypeStruct((N_NODE, H), f32)] * 2,
    )(u_ll, fc_ll, u_pl, fc_pl, u_pp, fc_pp, u_lp, fc_lp,
      h_l, h_p, w2i, w2t, b2i, b2t, gwa, gwb, gb,
      wih_l, whh_l, bih_l, bhh_l, wih_p, whh_p, bih_p, bhh_p)


def _merge_halves(lo, hi):
    return jnp.concatenate([lo[:HALF], hi[:N_NODE - HALF]], axis=0)


def kernel(h_l, h_p, edge_index_ll, edge_attr_ll, edge_weight_ll,
           edge_index_pp, edge_attr_pp, edge_weight_pp,
           edge_index_pl, edge_attr_pl, edge_weight_pl,
           inter_W1, inter_b1, inter_W2, inter_b2,
           intra_W1, intra_b1, intra_W2, intra_b2,
           gate_W, gate_b,
           gru_l_Wih, gru_l_Whh, gru_l_bih, gru_l_bhh,
           gru_p_Wih, gru_p_Whh, gru_p_bih, gru_p_bhh):
    wcat = jnp.concatenate([inter_W1[0:H], inter_W1[H + D_E:],
                            intra_W1[0:H], intra_W1[H + D_E:]], axis=1)
    wbi = inter_W1[H:H + D_E]
    wbt = intra_W1[H:H + D_E]
    b1i = inter_b1.reshape(1, H)
    b1t = intra_b1.reshape(1, H)

    a_l, b_l, c_l, d_l, a_p, b_p, c_p, d_p = _node_proj(h_l, h_p, wcat)
    p_ll, p_pp, q_pl, fs_ll, fs_pp, fs_pl = _edge_pre(
        edge_attr_ll, edge_attr_pp, edge_attr_pl,
        edge_weight_ll, edge_weight_pp, edge_weight_pl, wbi, wbt, b1i, b1t)

    ll0 = edge_index_ll[0].astype(i32)
    ll1 = edge_index_ll[1].astype(i32)
    pp0 = edge_index_pp[0].astype(i32)
    pp1 = edge_index_pp[1].astype(i32)
    pl0 = edge_index_pl[0].astype(i32)
    pl1 = edge_index_pl[1].astype(i32)

    u_ll = _merge_halves(*_sc_edge_pass(a_l, b_l, p_ll, fs_ll, ll0, ll1))
    u_pp = _merge_halves(*_sc_edge_pass(a_p, b_p, p_pp, fs_pp, pp0, pp1))
    u_pl = _merge_halves(*_sc_edge_pass(c_p, d_l, q_pl, fs_pl, pl0, pl1))
    u_lp = _merge_halves(*_sc_edge_pass(c_l, d_p, q_pl, fs_pl, pl1, pl0))

    fc_out = _sc_fc_pass(fs_ll, ll1, fs_pp, pp1, fs_pl, pl1, fs_pl, pl0)
    fc_ll = _merge_halves(fc_out[0], fc_out[1])
    fc_pp = _merge_halves(fc_out[2], fc_out[3])
    fc_pl = _merge_halves(fc_out[4], fc_out[5])
    fc_lp = _merge_halves(fc_out[6], fc_out[7])

    new_h_l, new_h_p = _node_post(
        u_ll, fc_ll, u_pl, fc_pl, u_pp, fc_pp, u_lp, fc_lp,
        h_l, h_p, inter_W2, intra_W2,
        inter_b2.reshape(1, H), intra_b2.reshape(1, H),
        gate_W[0:H], gate_W[H:], gate_b.reshape(1, 1),
        gru_l_Wih.T, gru_l_Whh.T, gru_l_bih.reshape(1, 3 * H),
        gru_l_bhh.reshape(1, 3 * H),
        gru_p_Wih.T, gru_p_Whh.T, gru_p_bih.reshape(1, 3 * H),
        gru_p_bhh.reshape(1, 3 * H))
    return (new_h_l, new_h_p)
